# baseline (device time: 149438 ns/iter reference)
import jax
import jax.numpy as jnp
from jax import lax
from jax.experimental import pallas as pl
from jax.experimental.pallas import tpu as pltpu

N_DEV = 16
M = 1024
N = 1024
CHUNK = M // N_DEV


def kernel(A, B):
    def body(a_ref, b_ref, out_ref, comm_ref,
             rs_send_sems, rs_recv_sems, ag_send_sems, ag_recv_sems):
        my = lax.axis_index("i")
        left = lax.rem(my - 1 + N_DEV, N_DEV)
        right = lax.rem(my + 1, N_DEV)

        barrier_sem = pltpu.get_barrier_semaphore()
        for nbr in (left, right):
            pl.semaphore_signal(barrier_sem, inc=1, device_id=(nbr,),
                                device_id_type=pl.DeviceIdType.MESH)
        pl.semaphore_wait(barrier_sem, 2)

        out_ref[:, :] = jnp.dot(a_ref[:, :], b_ref[:, :],
                                preferred_element_type=jnp.float32)

        for h in range(N_DEV - 1):
            send_chunk = lax.rem(my - h + 2 * N_DEV, N_DEV)
            recv_chunk = lax.rem(my - h - 1 + 2 * N_DEV, N_DEV)
            rdma = pltpu.make_async_remote_copy(
                src_ref=out_ref.at[pl.ds(send_chunk * CHUNK, CHUNK), :],
                dst_ref=comm_ref.at[h],
                send_sem=rs_send_sems.at[h],
                recv_sem=rs_recv_sems.at[h],
                device_id=(right,),
                device_id_type=pl.DeviceIdType.MESH,
            )
            rdma.start()
            rdma.wait()
            out_ref[pl.ds(recv_chunk * CHUNK, CHUNK), :] = (
                out_ref[pl.ds(recv_chunk * CHUNK, CHUNK), :] + comm_ref[h]
            )

        for h in range(N_DEV - 1):
            g = lax.rem(my + 1 - h + 2 * N_DEV, N_DEV)
            rdma = pltpu.make_async_remote_copy(
                src_ref=out_ref.at[pl.ds(g * CHUNK, CHUNK), :],
                dst_ref=out_ref.at[pl.ds(g * CHUNK, CHUNK), :],
                send_sem=ag_send_sems.at[h],
                recv_sem=ag_recv_sems.at[h],
                device_id=(right,),
                device_id_type=pl.DeviceIdType.MESH,
            )
            rdma.start()
            rdma.wait()

    return pl.pallas_call(
        body,
        out_shape=jax.ShapeDtypeStruct((M, N), jnp.float32),
        in_specs=[
            pl.BlockSpec(memory_space=pltpu.VMEM),
            pl.BlockSpec(memory_space=pltpu.VMEM),
        ],
        out_specs=pl.BlockSpec(memory_space=pltpu.VMEM),
        scratch_shapes=[
            pltpu.VMEM((N_DEV - 1, CHUNK, N), jnp.float32),
            pltpu.SemaphoreType.DMA((N_DEV - 1,)),
            pltpu.SemaphoreType.DMA((N_DEV - 1,)),
            pltpu.SemaphoreType.DMA((N_DEV - 1,)),
            pltpu.SemaphoreType.DMA((N_DEV - 1,)),
        ],
        compiler_params=pltpu.CompilerParams(collective_id=0),
    )(A, B)


# device time: 146620 ns/iter; 1.0192x vs baseline; 1.0192x over previous
import jax
import jax.numpy as jnp
from jax import lax
from jax.experimental import pallas as pl
from jax.experimental.pallas import tpu as pltpu

N_DEV = 16
M = 1024
N = 1024
CHUNK = M // N_DEV
HALF = N // 2
NH = N_DEV - 1


def kernel(A, B):
    def body(a_ref, b_ref, out_ref, commR_ref, commL_ref,
             rsR_send, rsR_recv, rsL_send, rsL_recv,
             agR_send, agR_recv, agL_send, agL_recv):
        my = lax.axis_index("i")
        left = lax.rem(my - 1 + N_DEV, N_DEV)
        right = lax.rem(my + 1, N_DEV)

        def rc(delta):
            return lax.rem(my + delta + 2 * N_DEV, N_DEV)

        def compute_chunk(delta):
            idx = rc(delta)
            out_ref[pl.ds(idx * CHUNK, CHUNK), :] = jnp.dot(
                a_ref[pl.ds(idx * CHUNK, CHUNK), :], b_ref[:, :],
                preferred_element_type=jnp.float32)

        barrier_sem = pltpu.get_barrier_semaphore()
        for nbr in (left, right):
            pl.semaphore_signal(barrier_sem, inc=1, device_id=(nbr,),
                                device_id_type=pl.DeviceIdType.MESH)
        pl.semaphore_wait(barrier_sem, 2)

        compute_chunk(0)

        for h in range(NH):
            sendR = pltpu.make_async_remote_copy(
                src_ref=out_ref.at[pl.ds(rc(-h) * CHUNK, CHUNK), pl.ds(0, HALF)],
                dst_ref=commR_ref.at[h],
                send_sem=rsR_send.at[h], recv_sem=rsR_recv.at[h],
                device_id=(right,), device_id_type=pl.DeviceIdType.MESH)
            sendL = pltpu.make_async_remote_copy(
                src_ref=out_ref.at[pl.ds(rc(h) * CHUNK, CHUNK), pl.ds(HALF, HALF)],
                dst_ref=commL_ref.at[h],
                send_sem=rsL_send.at[h], recv_sem=rsL_recv.at[h],
                device_id=(left,), device_id_type=pl.DeviceIdType.MESH)
            sendR.start()
            sendL.start()
            if h + 1 <= 7:
                compute_chunk(-(h + 1))
                compute_chunk(h + 1)
            elif h + 1 == 8:
                compute_chunk(8)
            sendR.wait()
            sendL.wait()
            iR = rc(-h - 1)
            out_ref[pl.ds(iR * CHUNK, CHUNK), pl.ds(0, HALF)] = (
                out_ref[pl.ds(iR * CHUNK, CHUNK), pl.ds(0, HALF)] + commR_ref[h])
            iL = rc(h + 1)
            out_ref[pl.ds(iL * CHUNK, CHUNK), pl.ds(HALF, HALF)] = (
                out_ref[pl.ds(iL * CHUNK, CHUNK), pl.ds(HALF, HALF)] + commL_ref[h])

        for h in range(NH):
            gR = rc(1 - h)
            sendR = pltpu.make_async_remote_copy(
                src_ref=out_ref.at[pl.ds(gR * CHUNK, CHUNK), pl.ds(0, HALF)],
                dst_ref=out_ref.at[pl.ds(gR * CHUNK, CHUNK), pl.ds(0, HALF)],
                send_sem=agR_send.at[h], recv_sem=agR_recv.at[h],
                device_id=(right,), device_id_type=pl.DeviceIdType.MESH)
            gL = rc(h - 1)
            sendL = pltpu.make_async_remote_copy(
                src_ref=out_ref.at[pl.ds(gL * CHUNK, CHUNK), pl.ds(HALF, HALF)],
                dst_ref=out_ref.at[pl.ds(gL * CHUNK, CHUNK), pl.ds(HALF, HALF)],
                send_sem=agL_send.at[h], recv_sem=agL_recv.at[h],
                device_id=(left,), device_id_type=pl.DeviceIdType.MESH)
            sendR.start()
            sendL.start()
            sendR.wait()
            sendL.wait()

    return pl.pallas_call(
        body,
        out_shape=jax.ShapeDtypeStruct((M, N), jnp.float32),
        in_specs=[
            pl.BlockSpec(memory_space=pltpu.VMEM),
            pl.BlockSpec(memory_space=pltpu.VMEM),
        ],
        out_specs=pl.BlockSpec(memory_space=pltpu.VMEM),
        scratch_shapes=[
            pltpu.VMEM((NH, CHUNK, HALF), jnp.float32),
            pltpu.VMEM((NH, CHUNK, HALF), jnp.float32),
            pltpu.SemaphoreType.DMA((NH,)),
            pltpu.SemaphoreType.DMA((NH,)),
            pltpu.SemaphoreType.DMA((NH,)),
            pltpu.SemaphoreType.DMA((NH,)),
            pltpu.SemaphoreType.DMA((NH,)),
            pltpu.SemaphoreType.DMA((NH,)),
            pltpu.SemaphoreType.DMA((NH,)),
            pltpu.SemaphoreType.DMA((NH,)),
        ],
        compiler_params=pltpu.CompilerParams(collective_id=0),
    )(A, B)


# device time: 77002 ns/iter; 1.9407x vs baseline; 1.9041x over previous
import jax
import jax.numpy as jnp
from jax import lax
from jax.experimental import pallas as pl
from jax.experimental.pallas import tpu as pltpu

N_DEV = 16
M = 1024
N = 1024
QROWS = 256
PROWS = 64
HALF = 512
NST = 12


def kernel(A, B):
    def body(a_ref, b_ref, out_ref, cA1_ref, cA2_ref, cB1_ref, cB2_ref,
             a_send, a_recv, b_send, b_recv):
        my = lax.axis_index("i")
        z4 = lax.div(my, 4)
        q4 = lax.rem(my, 4)

        def m4(v):
            return lax.rem(v + 8, 4)

        pr = z4 * 4 + m4(q4 + 1)
        plq = z4 * 4 + m4(q4 - 1)
        zr = m4(z4 + 1) * 4 + q4
        zl = m4(z4 - 1) * 4 + q4

        colsA = pl.ds(0, HALF)
        colsB = pl.ds(HALF, HALF)

        def qrows(qi):
            return pl.ds(qi * QROWS, QROWS)

        def prow(qi, pi):
            return pl.ds(qi * QROWS + pi * PROWS, PROWS)

        barrier_sem = pltpu.get_barrier_semaphore()
        for nbr in (pr, plq, zr, zl):
            pl.semaphore_signal(barrier_sem, inc=1, device_id=(nbr,),
                                device_id_type=pl.DeviceIdType.MESH)
        pl.semaphore_wait(barrier_sem, 4)

        out_ref[:, :] = jnp.dot(a_ref[:, :], b_ref[:, :],
                                preferred_element_type=jnp.float32)

        def step(s, rdmaA, rdmaB, accumA, accumB):
            rdmaA.start()
            rdmaB.start()
            rdmaA.wait()
            rdmaB.wait()
            if accumA is not None:
                accumA()
            if accumB is not None:
                accumB()

        for s in range(3):
            sqA, rqA = m4(q4 - s), m4(q4 - 1 - s)
            sqB, rqB = m4(z4 - s), m4(z4 - 1 - s)
            rdmaA = pltpu.make_async_remote_copy(
                src_ref=out_ref.at[qrows(sqA), colsA], dst_ref=cA1_ref.at[s],
                send_sem=a_send.at[s], recv_sem=a_recv.at[s],
                device_id=(pr,), device_id_type=pl.DeviceIdType.MESH)
            rdmaB = pltpu.make_async_remote_copy(
                src_ref=out_ref.at[qrows(sqB), colsB], dst_ref=cB1_ref.at[s],
                send_sem=b_send.at[s], recv_sem=b_recv.at[s],
                device_id=(zr,), device_id_type=pl.DeviceIdType.MESH)

            def accA(rqA=rqA, s=s):
                out_ref[qrows(rqA), colsA] = (
                    out_ref[qrows(rqA), colsA] + cA1_ref[s])

            def accB(rqB=rqB, s=s):
                out_ref[qrows(rqB), colsB] = (
                    out_ref[qrows(rqB), colsB] + cB1_ref[s])

            step(s, rdmaA, rdmaB, accA, accB)

        QA = m4(q4 + 1)
        QB = m4(z4 + 1)

        for t in range(3):
            s = 3 + t
            spA, rpA = m4(z4 - t), m4(z4 - 1 - t)
            spB, rpB = m4(q4 - t), m4(q4 - 1 - t)
            rdmaA = pltpu.make_async_remote_copy(
                src_ref=out_ref.at[prow(QA, spA), colsA], dst_ref=cA2_ref.at[t],
                send_sem=a_send.at[s], recv_sem=a_recv.at[s],
                device_id=(zr,), device_id_type=pl.DeviceIdType.MESH)
            rdmaB = pltpu.make_async_remote_copy(
                src_ref=out_ref.at[prow(QB, spB), colsB], dst_ref=cB2_ref.at[t],
                send_sem=b_send.at[s], recv_sem=b_recv.at[s],
                device_id=(pr,), device_id_type=pl.DeviceIdType.MESH)

            def accA(rpA=rpA, t=t):
                out_ref[prow(QA, rpA), colsA] = (
                    out_ref[prow(QA, rpA), colsA] + cA2_ref[t])

            def accB(rpB=rpB, t=t):
                out_ref[prow(QB, rpB), colsB] = (
                    out_ref[prow(QB, rpB), colsB] + cB2_ref[t])

            step(s, rdmaA, rdmaB, accA, accB)

        for t in range(3):
            s = 6 + t
            gpA = m4(z4 + 1 - t)
            gpB = m4(q4 + 1 - t)
            rdmaA = pltpu.make_async_remote_copy(
                src_ref=out_ref.at[prow(QA, gpA), colsA],
                dst_ref=out_ref.at[prow(QA, gpA), colsA],
                send_sem=a_send.at[s], recv_sem=a_recv.at[s],
                device_id=(zr,), device_id_type=pl.DeviceIdType.MESH)
            rdmaB = pltpu.make_async_remote_copy(
                src_ref=out_ref.at[prow(QB, gpB), colsB],
                dst_ref=out_ref.at[prow(QB, gpB), colsB],
                send_sem=b_send.at[s], recv_sem=b_recv.at[s],
                device_id=(pr,), device_id_type=pl.DeviceIdType.MESH)
            step(s, rdmaA, rdmaB, None, None)

        for t in range(3):
            s = 9 + t
            gqA = m4(q4 + 1 - t)
            gqB = m4(z4 + 1 - t)
            rdmaA = pltpu.make_async_remote_copy(
                src_ref=out_ref.at[qrows(gqA), colsA],
                dst_ref=out_ref.at[qrows(gqA), colsA],
                send_sem=a_send.at[s], recv_sem=a_recv.at[s],
                device_id=(pr,), device_id_type=pl.DeviceIdType.MESH)
            rdmaB = pltpu.make_async_remote_copy(
                src_ref=out_ref.at[qrows(gqB), colsB],
                dst_ref=out_ref.at[qrows(gqB), colsB],
                send_sem=b_send.at[s], recv_sem=b_recv.at[s],
                device_id=(zr,), device_id_type=pl.DeviceIdType.MESH)
            step(s, rdmaA, rdmaB, None, None)

    return pl.pallas_call(
        body,
        out_shape=jax.ShapeDtypeStruct((M, N), jnp.float32),
        in_specs=[
            pl.BlockSpec(memory_space=pltpu.VMEM),
            pl.BlockSpec(memory_space=pltpu.VMEM),
        ],
        out_specs=pl.BlockSpec(memory_space=pltpu.VMEM),
        scratch_shapes=[
            pltpu.VMEM((3, QROWS, HALF), jnp.float32),
            pltpu.VMEM((3, PROWS, HALF), jnp.float32),
            pltpu.VMEM((3, QROWS, HALF), jnp.float32),
            pltpu.VMEM((3, PROWS, HALF), jnp.float32),
            pltpu.SemaphoreType.DMA((NST,)),
            pltpu.SemaphoreType.DMA((NST,)),
            pltpu.SemaphoreType.DMA((NST,)),
            pltpu.SemaphoreType.DMA((NST,)),
        ],
        compiler_params=pltpu.CompilerParams(collective_id=0),
    )(A, B)
